# baseline (speedup 1.0000x reference)
import jax
import jax.numpy as jnp
from jax import lax
from jax.experimental import pallas as pl
from jax.experimental.pallas import tpu as pltpu
from jax.experimental.pallas import tpu_sc as plsc

B, K, C, T, TD = 8, 20000, 256, 1024, 256
H = C // 2
KP = 20480
BK = 1024
NROW = KP // 128


def _score_body(f_ref, xz_ref, w1_ref, b1_ref, w2_ref, b2_ref, s_ref):
    f = f_ref[0]
    h = lax.dot_general(w1_ref[...], f, (((0,), (0,)), ((), ())),
                        preferred_element_type=jnp.float32)
    h = jnp.maximum(h + b1_ref[...], 0.0)
    logit = lax.dot_general(w2_ref[...], h, (((0,), (0,)), ((), ())),
                            preferred_element_type=jnp.float32)
    logit = logit + b2_ref[0, 0]
    grasp = jax.nn.sigmoid(logit)
    xz = xz_ref[0]
    valid = jnp.sum(jnp.abs(xz), axis=0, keepdims=True) > 0.0
    s_ref[0] = jnp.where(valid, grasp, -jnp.inf)


def _scores(features, xyzT_pad, W1, b1, W2, b2):
    return pl.pallas_call(
        _score_body,
        grid=(B, KP // BK),
        in_specs=[
            pl.BlockSpec((1, C, BK), lambda b, k: (b, 0, k)),
            pl.BlockSpec((1, 3, BK), lambda b, k: (b, 0, k)),
            pl.BlockSpec((C, H), lambda b, k: (0, 0)),
            pl.BlockSpec((H, 1), lambda b, k: (0, 0)),
            pl.BlockSpec((H, 1), lambda b, k: (0, 0)),
            pl.BlockSpec((1, 1), lambda b, k: (0, 0)),
        ],
        out_specs=pl.BlockSpec((1, 1, BK), lambda b, k: (b, 0, k)),
        out_shape=jax.ShapeDtypeStruct((B, 1, KP), jnp.float32),
    )(features, xyzT_pad, W1, b1.reshape(H, 1), W2, b2.reshape(1, 1))


def _topk_body(s_ref, idx_ref):
    bits = lax.bitcast_convert_type(s_ref[0], jnp.int32)

    def bis(i, t):
        cand = jnp.bitwise_or(t, jnp.left_shift(jnp.int32(1), 30 - i))
        cnt = jnp.sum((bits >= cand).astype(jnp.float32))
        return jnp.where(cnt >= float(T), cand, t)

    t = lax.fori_loop(0, 31, bis, jnp.zeros((), jnp.int32))

    gt = (bits > t).astype(jnp.float32).reshape(NROW, 128)
    eq = (bits == t).astype(jnp.float32).reshape(NROW, 128)
    u128 = (lax.broadcasted_iota(jnp.int32, (128, 128), 0)
            < lax.broadcasted_iota(jnp.int32, (128, 128), 1)).astype(jnp.float32)
    urow = (lax.broadcasted_iota(jnp.int32, (NROW, NROW), 0)
            < lax.broadcasted_iota(jnp.int32, (NROW, NROW), 1)).astype(jnp.float32)
    lrow = (lax.broadcasted_iota(jnp.int32, (NROW, NROW), 0)
            > lax.broadcasted_iota(jnp.int32, (NROW, NROW), 1)).astype(jnp.float32)
    ones128 = jnp.ones((1, 128), jnp.float32)

    def excl_prefix(rows):
        within = lax.dot_general(rows, u128, (((1,), (0,)), ((), ())),
                                 preferred_element_type=jnp.float32,
                             precision=lax.Precision.HIGHEST)
        rs_row = lax.dot_general(ones128, rows, (((1,), (1,)), ((), ())),
                                 preferred_element_type=jnp.float32,
                             precision=lax.Precision.HIGHEST)
        before_row = lax.dot_general(rs_row, urow, (((1,), (0,)), ((), ())),
                                     preferred_element_type=jnp.float32,
                             precision=lax.Precision.HIGHEST)
        before_col = lax.dot_general(lrow, jnp.sum(rows, axis=1, keepdims=True),
                                     (((1,), (0,)), ((), ())),
                                     preferred_element_type=jnp.float32,
                             precision=lax.Precision.HIGHEST)
        return within + before_col, before_row

    tie_rank, _ = excl_prefix(eq)
    need = float(T) - jnp.sum(gt)
    sel = jnp.minimum(gt + eq * (tie_rank < need), 1.0)
    within, before_row = excl_prefix(sel)

    hi = lax.shift_right_logical(bits, 16).astype(jnp.float32).reshape(NROW, 128)
    lo = jnp.bitwise_and(bits, 0xFFFF).astype(jnp.float32).reshape(NROW, 128)

    slot = lax.broadcasted_iota(jnp.int32, (T, 1), 0).astype(jnp.float32)
    lane = lax.broadcasted_iota(jnp.int32, (1, 128), 1).astype(jnp.float32)
    rowid = lax.broadcasted_iota(jnp.int32, (1, NROW), 1).astype(jnp.float32)

    row_of = jnp.sum((before_row <= slot).astype(jnp.float32), axis=1,
                     keepdims=True) - 1.0
    rowsel = (row_of == rowid).astype(jnp.float32)
    stacked = jnp.concatenate([within, sel, hi, lo], axis=1)
    g = lax.dot_general(rowsel, stacked, (((1,), (0,)), ((), ())),
                        preferred_element_type=jnp.float32,
                             precision=lax.Precision.HIGHEST)
    match = (g[:, 0:128] == slot).astype(jnp.float32) * g[:, 128:256]
    k_col = jnp.sum(match * lane, axis=1, keepdims=True) + row_of * 128.0
    hi_col = jnp.sum(match * g[:, 256:384], axis=1, keepdims=True)
    lo_col = jnp.sum(match * g[:, 384:512], axis=1, keepdims=True)

    ident = (lax.broadcasted_iota(jnp.int32, (T, T), 0)
             == lax.broadcasted_iota(jnp.int32, (T, T), 1)).astype(jnp.float32)
    hilo = jnp.concatenate([hi_col, lo_col], axis=1)
    hilo_row = lax.dot_general(hilo, ident, (((0,), (0,)), ((), ())),
                               preferred_element_type=jnp.float32,
                             precision=lax.Precision.HIGHEST)
    hj = hilo_row[0:1]
    lj = hilo_row[1:2]
    jrow = lax.broadcasted_iota(jnp.int32, (1, T), 1).astype(jnp.float32)
    srow = jrow

    out_row = jnp.zeros((1, T), jnp.float32)
    for ib in range(T // 128):
        s = slice(ib * 128, (ib + 1) * 128)
        hi_i = hi_col[s]
        lo_i = lo_col[s]
        irow = (lax.broadcasted_iota(jnp.int32, (128, 1), 0)
                + ib * 128).astype(jnp.float32)
        beats = ((hj > hi_i) | ((hj == hi_i) & (lj > lo_i))
                 | ((hj == hi_i) & (lj == lo_i) & (jrow < irow)))
        rank_blk = jnp.sum(beats.astype(jnp.float32), axis=1, keepdims=True)
        onehot = (rank_blk == srow).astype(jnp.float32)
        out_row = out_row + lax.dot_general(
            k_col[s], onehot, (((0,), (0,)), ((), ())),
            preferred_element_type=jnp.float32,
                             precision=lax.Precision.HIGHEST)
    idx_ref[0, 0] = out_row[0].astype(jnp.int32)


def _topk(scores3):
    return pl.pallas_call(
        _topk_body,
        grid=(B,),
        in_specs=[pl.BlockSpec((1, 1, KP), lambda b: (b, 0, 0))],
        out_specs=pl.BlockSpec((1, 1, T), lambda b: (b, 0, 0)),
        out_shape=jax.ShapeDtypeStruct((B, 1, T), jnp.int32),
    )(scores3).reshape(B, T)


_NC = 2
_NS = 16
_NW = _NC * _NS
_PAIRS_PER_W = (B * C) // _NW
_XROWS_PER_W = (B * T) // _NW


def _gather_kernel(fe_hbm, xyz_hbm, tix_hbm, fsel_hbm, xsel_hbm,
                   tix_v, idx_v, row_v, xidx_v, xrow_v, sem, xsem):
    wid = lax.axis_index("s") * _NC + lax.axis_index("c")

    p0 = wid * _PAIRS_PER_W
    b0 = p0 // C
    pltpu.sync_copy(tix_hbm.at[b0], tix_v)

    def pair_step(j, _):
        p = p0 + j
        base = p * K

        def build(r, _):
            def lanes(v, _):
                idx_v[r, pl.ds(v * 16, 16)] = (
                    tix_v[pl.ds(r * 128 + v * 16, 16)] + base)
                return 0
            return lax.fori_loop(0, 8, lanes, 0)
        lax.fori_loop(0, T // 128, build, 0)

        def fire(r, _):
            pltpu.async_copy(fe_hbm.at[idx_v.at[r]],
                             row_v.at[pl.ds(r * 128, 128)], sem)
            return 0
        lax.fori_loop(0, T // 128, fire, 0)

        def drain(r, _):
            pltpu.make_async_copy(fe_hbm.at[idx_v.at[0]],
                                  row_v.at[pl.ds(0, 128)], sem).wait()
            return 0
        lax.fori_loop(0, T // 128, drain, 0)
        pltpu.sync_copy(row_v, fsel_hbm.at[pl.ds(p * T, T)])
        return 0

    lax.fori_loop(0, _PAIRS_PER_W, pair_step, 0)

    r0 = wid * _XROWS_PER_W
    s0 = r0 - b0 * T
    nch = _XROWS_PER_W // 128

    def xbuild(h, _):
        def xch(c, _):
            def xlan(v, _):
                xidx_v[c * nch + h, pl.ds(v * 16, 16)] = (
                    (tix_v[pl.ds(s0 + h * 128 + v * 16, 16)] + b0 * K) * 3 + c)
                return 0
            return lax.fori_loop(0, 8, xlan, 0)
        return lax.fori_loop(0, 3, xch, 0)
    lax.fori_loop(0, nch, xbuild, 0)

    def xfire(cc, _):
        pltpu.async_copy(xyz_hbm.at[xidx_v.at[cc]],
                         xrow_v.at[pl.ds(cc * 128, 128)], xsem)
        return 0
    lax.fori_loop(0, 3 * nch, xfire, 0)

    def xdrain(cc, _):
        pltpu.make_async_copy(xyz_hbm.at[xidx_v.at[0]],
                              xrow_v.at[pl.ds(0, 128)], xsem).wait()
        return 0
    lax.fori_loop(0, 3 * nch, xdrain, 0)

    def xout(c, _):
        pltpu.sync_copy(
            xrow_v.at[pl.ds(c * _XROWS_PER_W, _XROWS_PER_W)],
            xsel_hbm.at[pl.ds(c * (B * T) + r0, _XROWS_PER_W)])
        return 0
    lax.fori_loop(0, 3, xout, 0)


def _sc_gather(fe_flat, xyz_rows, top_idx):
    mesh = plsc.VectorSubcoreMesh(core_axis_name="c", subcore_axis_name="s")
    f = pl.kernel(
        _gather_kernel,
        mesh=mesh,
        out_type=(jax.ShapeDtypeStruct((B * C * T,), jnp.float32),
                  jax.ShapeDtypeStruct((3 * B * T,), jnp.float32)),
        scratch_types=[
            pltpu.VMEM((T,), jnp.int32),
            pltpu.VMEM((T // 128, 128), jnp.int32),
            pltpu.VMEM((T,), jnp.float32),
            pltpu.VMEM((3 * (_XROWS_PER_W // 128), 128), jnp.int32),
            pltpu.VMEM((3 * _XROWS_PER_W,), jnp.float32),
            pltpu.SemaphoreType.DMA,
            pltpu.SemaphoreType.DMA,
        ],
    )
    return f(fe_flat, xyz_rows, top_idx)


def _out_body(fs_ref, w3_ref, b3_ref, g_ref, be_ref, o_ref):
    fs = fs_ref[0]
    tt = lax.dot_general(w3_ref[...], fs, (((0,), (0,)), ((), ())),
                         preferred_element_type=jnp.float32)
    tt = tt + b3_ref[...]
    mu = jnp.mean(tt, axis=0, keepdims=True)
    d = tt - mu
    var = jnp.mean(d * d, axis=0, keepdims=True)
    o_ref[0] = d / jnp.sqrt(var + 1e-5) * g_ref[...] + be_ref[...]


def _out_proj(fselT, W3, b3, gamma, beta):
    TB = 256
    return pl.pallas_call(
        _out_body,
        grid=(B, T // TB),
        in_specs=[
            pl.BlockSpec((1, C, TB), lambda b, t: (b, 0, t)),
            pl.BlockSpec((C, TD), lambda b, t: (0, 0)),
            pl.BlockSpec((TD, 1), lambda b, t: (0, 0)),
            pl.BlockSpec((TD, 1), lambda b, t: (0, 0)),
            pl.BlockSpec((TD, 1), lambda b, t: (0, 0)),
        ],
        out_specs=pl.BlockSpec((1, TD, TB), lambda b, t: (b, 0, t)),
        out_shape=jax.ShapeDtypeStruct((B, TD, T), jnp.float32),
    )(fselT, W3, b3.reshape(TD, 1), gamma.reshape(TD, 1), beta.reshape(TD, 1))


@jax.jit
def kernel(xyz, features, W1, b1, W2, b2, W3, b3, gamma, beta):
    xyzT = jnp.transpose(xyz, (0, 2, 1))
    xyzT_pad = jnp.pad(xyzT, ((0, 0), (0, 0), (0, KP - K)))
    scores = _scores(features, xyzT_pad, W1, b1, W2, b2)
    top_idx = _topk(scores)

    fe_flat = features.reshape(B * C * K)
    fsel_flat, xsel = _sc_gather(fe_flat, xyz.reshape(B * K * 3), top_idx)
    fselT = fsel_flat.reshape(B, C, T)

    feat_out = _out_proj(fselT, W3, b3, gamma, beta)
    xyz_sel = jnp.transpose(xsel.reshape(3, B, T), (1, 2, 0))
    return xyz_sel, feat_out
